# baseline (speedup 1.0000x reference)
import functools

import jax
import jax.numpy as jnp
from jax import lax
from jax.experimental import pallas as pl
from jax.experimental.pallas import tpu as pltpu
from jax.experimental.pallas import tpu_sc as plsc

N = 10000
E = 320000
T = 4
DIN = 128
DH = 64
NPAD = 10240
NC = 2
NS = 16
NW = NC * NS
EB = 128
B_CNT = 80
EPAD = NW * EB * B_CNT
ZRPS = NPAD // NS

_F32 = jnp.float32


def _deg_body(dstw, out, dst_v, oneh_v, zero_v, a0, a1, dsem):
    c = lax.axis_index("c")
    s = lax.axis_index("s")
    wid = s * NC + c
    accs = [a0, a1]

    lanes = lax.iota(jnp.int32, 16)
    e0 = jnp.where(lanes == 0, 1.0, 0.0).astype(_F32)
    z16 = jnp.zeros((16,), _F32)

    def _fill_oneh(i, carry):
        oneh_v[i, :] = e0
        return carry

    lax.fori_loop(0, EB, _fill_oneh, 0)

    def _fill_zero(i, carry):
        zero_v[i, :] = z16
        return carry

    lax.fori_loop(0, ZRPS, _fill_zero, 0)

    for phase in range(2):
        for k in range(2):
            pltpu.sync_copy(zero_v, accs[k].at[pl.ds(s * ZRPS, ZRPS)])
        plsc.subcore_barrier()

        for k in range(2):
            t = phase * 2 + k
            pltpu.sync_copy(dstw.at[t, wid], dst_v)
            acc = accs[k]

            def _b(g, carry):
                for j in range(8):
                    pltpu.async_copy(oneh_v, acc.at[dst_v.at[g * 8 + j]],
                                     dsem, add=True)
                for j in range(8):
                    pltpu.make_async_copy(oneh_v, acc.at[dst_v.at[g * 8 + j]],
                                          dsem).wait()
                return carry

            lax.fori_loop(0, B_CNT // 8, _b, 0)
        plsc.subcore_barrier()

        for k in range(2):
            t = phase * 2 + k
            pltpu.sync_copy(accs[k].at[pl.ds(s * ZRPS, ZRPS)],
                            out.at[t, c, pl.ds(s * ZRPS, ZRPS)])
        plsc.subcore_barrier()


@functools.cache
def _deg_kernel():
    return functools.partial(
        pl.kernel,
        mesh=plsc.VectorSubcoreMesh(core_axis_name="c", subcore_axis_name="s"),
        compiler_params=pltpu.CompilerParams(use_tc_tiling_on_sc=False),
        out_type=jax.ShapeDtypeStruct((T, NC, NPAD, 16), _F32),
        scratch_types=[
            pltpu.VMEM((B_CNT, EB), jnp.int32),
            pltpu.VMEM((EB, 16), _F32),
            pltpu.VMEM((ZRPS, 16), _F32),
            pltpu.VMEM_SHARED((NPAD, 16), _F32),
            pltpu.VMEM_SHARED((NPAD, 16), _F32),
            pltpu.SemaphoreType.DMA,
        ],
    )(_deg_body)


def _prop_body(hs, srcw, dstw, out, src_v, dst_v, r0, r1, r2, hs_sh, acc,
               g0, g1, g2, s0, s1, s2):
    c = lax.axis_index("c")
    s = lax.axis_index("s")
    wid = s * NC + c
    rows = [r0, r1, r2]
    gsems = [g0, g1, g2]
    ssems = [s0, s1, s2]

    z16 = jnp.zeros((16,), _F32)

    def _fill_zero(i, carry):
        for j in range(DH // 16):
            r0[i, pl.ds(j * 16, 16)] = z16
        return carry

    lax.fori_loop(0, EB, _fill_zero, 0)

    pltpu.sync_copy(srcw.at[wid], src_v)
    pltpu.sync_copy(dstw.at[wid], dst_v)

    pltpu.sync_copy(hs.at[pl.ds(s * ZRPS, ZRPS)],
                    hs_sh.at[pl.ds(s * ZRPS, ZRPS)])
    for k in range(ZRPS // EB):
        pltpu.sync_copy(r0, acc.at[pl.ds(s * ZRPS + k * EB, EB)])
    plsc.subcore_barrier()

    def _fire_g(k, slot):
        pltpu.async_copy(hs_sh.at[src_v.at[k]], rows[slot], gsems[slot])

    def _wait_g(k, slot):
        pltpu.make_async_copy(hs_sh.at[src_v.at[k]], rows[slot],
                              gsems[slot]).wait()

    def _fire_s(k, slot):
        pltpu.async_copy(rows[slot], acc.at[dst_v.at[k]], ssems[slot],
                         add=True)

    def _wait_s(k, slot):
        pltpu.make_async_copy(rows[slot], acc.at[dst_v.at[k]],
                              ssems[slot]).wait()

    _fire_g(0, 0)
    _fire_g(1, 1)
    _fire_g(2, 2)
    _wait_g(0, 0)
    _fire_s(0, 0)
    _wait_s(0, 0)
    _fire_g(3, 0)
    _wait_g(1, 1)
    _fire_s(1, 1)

    def _grp(m, carry):
        k0 = m * 3 + 2
        for j in range(3):
            k = k0 + j
            slot = (2 + j) % 3
            _wait_s(k - 1, (slot + 2) % 3)
            _fire_g(k + 2, (slot + 2) % 3)
            _wait_g(k, slot)
            _fire_s(k, slot)
        return carry

    lax.fori_loop(0, (B_CNT - 5) // 3, _grp, 0)

    k = B_CNT - 3
    _wait_s(k - 1, (k + 2) % 3)
    _fire_g(k + 2, (k + 2) % 3)
    _wait_g(k, k % 3)
    _fire_s(k, k % 3)
    for k in (B_CNT - 2, B_CNT - 1):
        _wait_s(k - 1, (k - 1) % 3)
        _wait_g(k, k % 3)
        _fire_s(k, k % 3)
    _wait_s(B_CNT - 1, (B_CNT - 1) % 3)
    plsc.subcore_barrier()

    pltpu.sync_copy(acc.at[pl.ds(s * ZRPS, ZRPS)],
                    out.at[c, pl.ds(s * ZRPS, ZRPS)])


@functools.cache
def _prop_kernel():
    return functools.partial(
        pl.kernel,
        mesh=plsc.VectorSubcoreMesh(core_axis_name="c", subcore_axis_name="s"),
        compiler_params=pltpu.CompilerParams(use_tc_tiling_on_sc=False),
        out_type=jax.ShapeDtypeStruct((NC, NPAD, DH), _F32),
        scratch_types=[
            pltpu.VMEM((B_CNT, EB), jnp.int32),
            pltpu.VMEM((B_CNT, EB), jnp.int32),
            pltpu.VMEM((EB, DH), _F32),
            pltpu.VMEM((EB, DH), _F32),
            pltpu.VMEM((EB, DH), _F32),
            pltpu.VMEM_SHARED((NPAD, DH), _F32),
            pltpu.VMEM_SHARED((NPAD, DH), _F32),
            pltpu.SemaphoreType.DMA,
            pltpu.SemaphoreType.DMA,
            pltpu.SemaphoreType.DMA,
            pltpu.SemaphoreType.DMA,
            pltpu.SemaphoreType.DMA,
            pltpu.SemaphoreType.DMA,
        ],
    )(_prop_body)


BLK = 2048
GRID = NPAD // BLK


def _dinv(d0_ref, d1_ref):
    deg = d0_ref[:, 0:1] + d1_ref[:, 0:1] + 1.0
    return lax.rsqrt(deg)


def _s1_body(x_ref, w_ref, d0_ref, d1_ref, o_ref):
    dinv = _dinv(d0_ref, d1_ref)
    h = jnp.dot(x_ref[...], w_ref[...], preferred_element_type=_F32)
    o_ref[...] = h * dinv


def _s2_body(p0_ref, p1_ref, hs_ref, b_ref, w_ref, d0_ref, d1_ref, o_ref):
    dinv = _dinv(d0_ref, d1_ref)
    agg = p0_ref[...] + p1_ref[...] + hs_ref[...]
    cur = jnp.maximum(dinv * agg + b_ref[0:1, :], 0.0)
    o_ref[...] = jnp.dot(cur, w_ref[...], preferred_element_type=_F32) * dinv


def _gru_body(p0_ref, p1_ref, hs_ref, bh_ref, hid_ref,
              wua_ref, wub_ref, wra_ref, wrb_ref, wca_ref, wcb_ref,
              bu_ref, br_ref, bc_ref, wn_ref, bn_ref, d0_ref, d1_ref,
              hidn_ref, o_ref, *, scale_out):
    dinv = _dinv(d0_ref, d1_ref)
    cur = dinv * (p0_ref[...] + p1_ref[...] + hs_ref[...]) + bh_ref[0:1, :]
    hid = hid_ref[...]
    dot = functools.partial(jnp.dot, preferred_element_type=_F32)
    u = jax.nn.sigmoid(dot(cur, wua_ref[...]) + dot(hid, wub_ref[...])
                       + bu_ref[0:1, :])
    r = jax.nn.sigmoid(dot(cur, wra_ref[...]) + dot(hid, wrb_ref[...])
                       + br_ref[0:1, :])
    cc = jnp.tanh(dot(cur, wca_ref[...]) + dot(r * hid, wcb_ref[...])
                  + bc_ref[0:1, :])
    hidn = (1.0 - u) * hid + u * cc
    hidn_ref[...] = hidn
    nxt = dot(hidn, wn_ref[...])
    if scale_out:
        o_ref[...] = nxt * dinv + bn_ref[0:1, :]
    else:
        o_ref[...] = nxt + bn_ref[0:1, :]


def _row_spec(cols):
    return pl.BlockSpec((BLK, cols), lambda i: (i, 0))


def _full_spec(rows, cols):
    return pl.BlockSpec((rows, cols), lambda i: (0, 0))


_s1_call = pl.pallas_call(
    _s1_body,
    grid=(GRID,),
    in_specs=[_row_spec(DIN), _full_spec(DIN, DH), _row_spec(16),
              _row_spec(16)],
    out_specs=_row_spec(DH),
    out_shape=jax.ShapeDtypeStruct((NPAD, DH), _F32),
)

_s2_call = pl.pallas_call(
    _s2_body,
    grid=(GRID,),
    in_specs=[_row_spec(DH), _row_spec(DH), _row_spec(DH),
              _full_spec(8, DH), _full_spec(DH, DH), _row_spec(16),
              _row_spec(16)],
    out_specs=_row_spec(DH),
    out_shape=jax.ShapeDtypeStruct((NPAD, DH), _F32),
)


def _gru_call(scale_out):
    return pl.pallas_call(
        functools.partial(_gru_body, scale_out=scale_out),
        grid=(GRID,),
        in_specs=[_row_spec(DH), _row_spec(DH), _row_spec(DH),
                  _full_spec(8, DH), _row_spec(DH)]
                 + [_full_spec(DH, DH)] * 6
                 + [_full_spec(8, DH)] * 3
                 + [_full_spec(DH, DH), _full_spec(8, DH),
                    _row_spec(16), _row_spec(16)],
        out_specs=[_row_spec(DH), _row_spec(DH)],
        out_shape=[jax.ShapeDtypeStruct((NPAD, DH), _F32),
                   jax.ShapeDtypeStruct((NPAD, DH), _F32)],
    )


_gru_mid = _gru_call(True)
_gru_last = _gru_call(False)


def _bias8(b):
    return jnp.broadcast_to(b.reshape(1, DH), (8, DH))


def kernel(x_sequence, edge_index_sequence,
           W_in_0, b_in_0, W_hid_0, b_hid_0, Wu_0, bu_0, Wr_0, br_0,
           Wc_0, bc_0,
           W_in_1, b_in_1, W_hid_1, b_hid_1, Wu_1, bu_1, Wr_1, br_1,
           Wc_1, bc_1, W_out, b_out):
    src = edge_index_sequence[:, 0, :]
    dst = edge_index_sequence[:, 1, :]
    npd = EPAD - E
    pad_src = jnp.broadcast_to(
        (jnp.arange(npd, dtype=jnp.int32) % 240)[None, :], (T, npd))
    pad_dst = jnp.broadcast_to(
        (N + jnp.arange(npd, dtype=jnp.int32) % (NPAD - N))[None, :], (T, npd))
    srcw = jnp.concatenate([src, pad_src], axis=1).reshape(T, NW, B_CNT, EB)
    dstw = jnp.concatenate([dst, pad_dst], axis=1).reshape(T, NW, B_CNT, EB)

    degp = _deg_kernel()(dstw)

    b_in0 = _bias8(b_in_0)
    b_hid0 = _bias8(b_hid_0)
    b_in1 = _bias8(b_in_1)
    b_hid1 = _bias8(b_hid_1)
    bu0, br0, bc0 = _bias8(bu_0), _bias8(br_0), _bias8(bc_0)
    bu1, br1, bc1 = _bias8(bu_1), _bias8(br_1), _bias8(bc_1)
    bz = jnp.zeros((8, DH), _F32)
    b_out8 = _bias8(b_out)

    Wu0a, Wu0b = Wu_0[:DH], Wu_0[DH:]
    Wr0a, Wr0b = Wr_0[:DH], Wr_0[DH:]
    Wc0a, Wc0b = Wc_0[:DH], Wc_0[DH:]
    Wu1a, Wu1b = Wu_1[:DH], Wu_1[DH:]
    Wr1a, Wr1b = Wr_1[:DH], Wr_1[DH:]
    Wc1a, Wc1b = Wc_1[:DH], Wc_1[DH:]

    xpad = jnp.concatenate(
        [x_sequence, jnp.zeros((T, NPAD - N, DIN), _F32)], axis=1)
    hid0 = jnp.zeros((NPAD, DH), _F32)
    hid1 = jnp.zeros((NPAD, DH), _F32)
    outs = []
    for t in range(T):
        d0t = degp[t, 0]
        d1t = degp[t, 1]
        st = srcw[t]
        dt = dstw[t]

        hs1 = _s1_call(xpad[t], W_in_0, d0t, d1t)
        p = _prop_kernel()(hs1, st, dt)
        hs2 = _s2_call(p[0], p[1], hs1, b_in0, W_hid_0, d0t, d1t)
        p = _prop_kernel()(hs2, st, dt)
        hid0, hs3 = _gru_mid(p[0], p[1], hs2, b_hid0, hid0,
                             Wu0a, Wu0b, Wr0a, Wr0b, Wc0a, Wc0b,
                             bu0, br0, bc0, W_in_1, bz, d0t, d1t)
        p = _prop_kernel()(hs3, st, dt)
        hs4 = _s2_call(p[0], p[1], hs3, b_in1, W_hid_1, d0t, d1t)
        p = _prop_kernel()(hs4, st, dt)
        hid1, out_t = _gru_last(p[0], p[1], hs4, b_hid1, hid1,
                                Wu1a, Wu1b, Wr1a, Wr1b, Wc1a, Wc1b,
                                bu1, br1, bc1, W_out, b_out8, d0t, d1t)
        outs.append(out_t[:N])
    return jnp.stack(outs)


# speedup vs baseline: 1.1648x; 1.1648x over previous
import functools

import jax
import jax.numpy as jnp
from jax import lax
from jax.experimental import pallas as pl
from jax.experimental.pallas import tpu as pltpu
from jax.experimental.pallas import tpu_sc as plsc

N = 10000
E = 320000
T = 4
DIN = 128
DH = 64
NPAD = 10240
NC = 2
NS = 16
NW = NC * NS
EB = 128
B_CNT = 80
EPAD = NW * EB * B_CNT
ZRPS = NPAD // NS

_F32 = jnp.float32


def _deg_body(dstw, out, dst_v, oneh_v, zero_v, a0, a1, dsem):
    c = lax.axis_index("c")
    s = lax.axis_index("s")
    wid = s * NC + c
    accs = [a0, a1]

    lanes = lax.iota(jnp.int32, 16)
    e0 = jnp.where(lanes == 0, 1.0, 0.0).astype(_F32)
    z16 = jnp.zeros((16,), _F32)

    def _fill_oneh(i, carry):
        oneh_v[i, :] = e0
        return carry

    lax.fori_loop(0, EB, _fill_oneh, 0)

    def _fill_zero(i, carry):
        zero_v[i, :] = z16
        return carry

    lax.fori_loop(0, ZRPS, _fill_zero, 0)

    for phase in range(2):
        for k in range(2):
            pltpu.sync_copy(zero_v, accs[k].at[pl.ds(s * ZRPS, ZRPS)])
        plsc.subcore_barrier()

        for k in range(2):
            t = phase * 2 + k
            pltpu.sync_copy(dstw.at[t, wid], dst_v)
            acc = accs[k]

            def _b(g, carry):
                for j in range(8):
                    pltpu.async_copy(oneh_v, acc.at[dst_v.at[g * 8 + j]],
                                     dsem, add=True)
                for j in range(8):
                    pltpu.make_async_copy(oneh_v, acc.at[dst_v.at[g * 8 + j]],
                                          dsem).wait()
                return carry

            lax.fori_loop(0, B_CNT // 8, _b, 0)
        plsc.subcore_barrier()

        for k in range(2):
            t = phase * 2 + k
            pltpu.sync_copy(accs[k].at[pl.ds(s * ZRPS, ZRPS)],
                            out.at[t, c, pl.ds(s * ZRPS, ZRPS)])
        plsc.subcore_barrier()


@functools.cache
def _deg_kernel():
    return functools.partial(
        pl.kernel,
        mesh=plsc.VectorSubcoreMesh(core_axis_name="c", subcore_axis_name="s"),
        compiler_params=pltpu.CompilerParams(use_tc_tiling_on_sc=False),
        out_type=jax.ShapeDtypeStruct((T, NC, NPAD, 16), _F32),
        scratch_types=[
            pltpu.VMEM((B_CNT, EB), jnp.int32),
            pltpu.VMEM((EB, 16), _F32),
            pltpu.VMEM((ZRPS, 16), _F32),
            pltpu.VMEM_SHARED((NPAD, 16), _F32),
            pltpu.VMEM_SHARED((NPAD, 16), _F32),
            pltpu.SemaphoreType.DMA,
        ],
    )(_deg_body)


def _prop_body(hs, srcw, dstw, out, src_v, dst_v, r0, r1, r2, r3, acc,
               g0, g1, g2, g3, s0, s1, s2, s3):
    c = lax.axis_index("c")
    s = lax.axis_index("s")
    wid = s * NC + c
    rows = [r0, r1, r2, r3]
    gsems = [g0, g1, g2, g3]
    ssems = [s0, s1, s2, s3]

    z16 = jnp.zeros((16,), _F32)

    def _fill_zero(i, carry):
        for j in range(DH // 16):
            r0[i, pl.ds(j * 16, 16)] = z16
        return carry

    lax.fori_loop(0, EB, _fill_zero, 0)

    pltpu.sync_copy(srcw.at[wid], src_v)
    pltpu.sync_copy(dstw.at[wid], dst_v)

    for k in range(ZRPS // EB):
        pltpu.sync_copy(r0, acc.at[pl.ds(s * ZRPS + k * EB, EB)])

    def _fire_g(k, slot):
        pltpu.async_copy(hs.at[src_v.at[k]], rows[slot], gsems[slot])

    def _wait_g(k, slot):
        pltpu.make_async_copy(hs.at[src_v.at[k]], rows[slot],
                              gsems[slot]).wait()

    def _fire_s(k, slot):
        pltpu.async_copy(rows[slot], acc.at[dst_v.at[k]], ssems[slot],
                         add=True)

    def _wait_s(k, slot):
        pltpu.make_async_copy(rows[slot], acc.at[dst_v.at[k]],
                              ssems[slot]).wait()

    _fire_g(0, 0)
    _fire_g(1, 1)
    plsc.subcore_barrier()

    _fire_g(2, 2)
    _wait_g(0, 0)
    _fire_s(0, 0)
    _fire_g(3, 3)
    _wait_g(1, 1)
    _fire_s(1, 1)
    for k in (2, 3):
        slot = k % 4
        _wait_s(k - 2, (k + 2) % 4)
        _fire_g(k + 2, (k + 2) % 4)
        _wait_g(k, slot)
        _fire_s(k, slot)

    def _grp(g, carry):
        k0 = g * 4
        for j in range(4):
            k = k0 + j
            _wait_s(k - 2, (j + 2) % 4)
            _fire_g(k + 2, (j + 2) % 4)
            _wait_g(k, j)
            _fire_s(k, j)
        return carry

    lax.fori_loop(1, (B_CNT - 4) // 4, _grp, 0)

    for k in (B_CNT - 4, B_CNT - 3):
        slot = k % 4
        _wait_s(k - 2, (k + 2) % 4)
        _fire_g(k + 2, (k + 2) % 4)
        _wait_g(k, slot)
        _fire_s(k, slot)
    for k in (B_CNT - 2, B_CNT - 1):
        slot = k % 4
        _wait_s(k - 2, (k + 2) % 4)
        _wait_g(k, slot)
        _fire_s(k, slot)
    _wait_s(B_CNT - 2, (B_CNT - 2) % 4)
    _wait_s(B_CNT - 1, (B_CNT - 1) % 4)
    plsc.subcore_barrier()

    pltpu.sync_copy(acc.at[pl.ds(s * ZRPS, ZRPS)],
                    out.at[c, pl.ds(s * ZRPS, ZRPS)])


@functools.cache
def _prop_kernel():
    return functools.partial(
        pl.kernel,
        mesh=plsc.VectorSubcoreMesh(core_axis_name="c", subcore_axis_name="s"),
        compiler_params=pltpu.CompilerParams(use_tc_tiling_on_sc=False),
        out_type=jax.ShapeDtypeStruct((NC, NPAD, DH), _F32),
        scratch_types=[
            pltpu.VMEM((B_CNT, EB), jnp.int32),
            pltpu.VMEM((B_CNT, EB), jnp.int32),
            pltpu.VMEM((EB, DH), _F32),
            pltpu.VMEM((EB, DH), _F32),
            pltpu.VMEM((EB, DH), _F32),
            pltpu.VMEM((EB, DH), _F32),
            pltpu.VMEM_SHARED((NPAD, DH), _F32),
            pltpu.SemaphoreType.DMA,
            pltpu.SemaphoreType.DMA,
            pltpu.SemaphoreType.DMA,
            pltpu.SemaphoreType.DMA,
            pltpu.SemaphoreType.DMA,
            pltpu.SemaphoreType.DMA,
            pltpu.SemaphoreType.DMA,
            pltpu.SemaphoreType.DMA,
        ],
    )(_prop_body)


BLK = 2048
GRID = NPAD // BLK


def _dinv(d0_ref, d1_ref):
    deg = d0_ref[:, 0:1] + d1_ref[:, 0:1] + 1.0
    return lax.rsqrt(deg)


def _s1_body(x_ref, w_ref, d0_ref, d1_ref, o_ref):
    dinv = _dinv(d0_ref, d1_ref)
    h = jnp.dot(x_ref[...], w_ref[...], preferred_element_type=_F32)
    o_ref[...] = h * dinv


def _s2_body(p0_ref, p1_ref, hs_ref, b_ref, w_ref, d0_ref, d1_ref, o_ref):
    dinv = _dinv(d0_ref, d1_ref)
    agg = p0_ref[...] + p1_ref[...] + hs_ref[...]
    cur = jnp.maximum(dinv * agg + b_ref[0:1, :], 0.0)
    o_ref[...] = jnp.dot(cur, w_ref[...], preferred_element_type=_F32) * dinv


def _gru_body(p0_ref, p1_ref, hs_ref, bh_ref, hid_ref,
              wua_ref, wub_ref, wra_ref, wrb_ref, wca_ref, wcb_ref,
              bu_ref, br_ref, bc_ref, wn_ref, bn_ref, d0_ref, d1_ref,
              hidn_ref, o_ref, *, scale_out):
    dinv = _dinv(d0_ref, d1_ref)
    cur = dinv * (p0_ref[...] + p1_ref[...] + hs_ref[...]) + bh_ref[0:1, :]
    hid = hid_ref[...]
    dot = functools.partial(jnp.dot, preferred_element_type=_F32)
    u = jax.nn.sigmoid(dot(cur, wua_ref[...]) + dot(hid, wub_ref[...])
                       + bu_ref[0:1, :])
    r = jax.nn.sigmoid(dot(cur, wra_ref[...]) + dot(hid, wrb_ref[...])
                       + br_ref[0:1, :])
    cc = jnp.tanh(dot(cur, wca_ref[...]) + dot(r * hid, wcb_ref[...])
                  + bc_ref[0:1, :])
    hidn = (1.0 - u) * hid + u * cc
    hidn_ref[...] = hidn
    nxt = dot(hidn, wn_ref[...])
    if scale_out:
        o_ref[...] = nxt * dinv + bn_ref[0:1, :]
    else:
        o_ref[...] = nxt + bn_ref[0:1, :]


def _row_spec(cols):
    return pl.BlockSpec((BLK, cols), lambda i: (i, 0))


def _full_spec(rows, cols):
    return pl.BlockSpec((rows, cols), lambda i: (0, 0))


_s1_call = pl.pallas_call(
    _s1_body,
    grid=(GRID,),
    in_specs=[_row_spec(DIN), _full_spec(DIN, DH), _row_spec(16),
              _row_spec(16)],
    out_specs=_row_spec(DH),
    out_shape=jax.ShapeDtypeStruct((NPAD, DH), _F32),
)

_s2_call = pl.pallas_call(
    _s2_body,
    grid=(GRID,),
    in_specs=[_row_spec(DH), _row_spec(DH), _row_spec(DH),
              _full_spec(8, DH), _full_spec(DH, DH), _row_spec(16),
              _row_spec(16)],
    out_specs=_row_spec(DH),
    out_shape=jax.ShapeDtypeStruct((NPAD, DH), _F32),
)


def _gru_call(scale_out):
    return pl.pallas_call(
        functools.partial(_gru_body, scale_out=scale_out),
        grid=(GRID,),
        in_specs=[_row_spec(DH), _row_spec(DH), _row_spec(DH),
                  _full_spec(8, DH), _row_spec(DH)]
                 + [_full_spec(DH, DH)] * 6
                 + [_full_spec(8, DH)] * 3
                 + [_full_spec(DH, DH), _full_spec(8, DH),
                    _row_spec(16), _row_spec(16)],
        out_specs=[_row_spec(DH), _row_spec(DH)],
        out_shape=[jax.ShapeDtypeStruct((NPAD, DH), _F32),
                   jax.ShapeDtypeStruct((NPAD, DH), _F32)],
    )


_gru_mid = _gru_call(True)
_gru_last = _gru_call(False)


def _bias8(b):
    return jnp.broadcast_to(b.reshape(1, DH), (8, DH))


def kernel(x_sequence, edge_index_sequence,
           W_in_0, b_in_0, W_hid_0, b_hid_0, Wu_0, bu_0, Wr_0, br_0,
           Wc_0, bc_0,
           W_in_1, b_in_1, W_hid_1, b_hid_1, Wu_1, bu_1, Wr_1, br_1,
           Wc_1, bc_1, W_out, b_out):
    src = edge_index_sequence[:, 0, :]
    dst = edge_index_sequence[:, 1, :]
    npd = EPAD - E
    pad_src = jnp.broadcast_to(
        (jnp.arange(npd, dtype=jnp.int32) % 240)[None, :], (T, npd))
    pad_dst = jnp.broadcast_to(
        (N + jnp.arange(npd, dtype=jnp.int32) % (NPAD - N))[None, :], (T, npd))
    srcw = jnp.concatenate([src, pad_src], axis=1).reshape(T, NW, B_CNT, EB)
    dstw = jnp.concatenate([dst, pad_dst], axis=1).reshape(T, NW, B_CNT, EB)

    degp = _deg_kernel()(dstw)

    b_in0 = _bias8(b_in_0)
    b_hid0 = _bias8(b_hid_0)
    b_in1 = _bias8(b_in_1)
    b_hid1 = _bias8(b_hid_1)
    bu0, br0, bc0 = _bias8(bu_0), _bias8(br_0), _bias8(bc_0)
    bu1, br1, bc1 = _bias8(bu_1), _bias8(br_1), _bias8(bc_1)
    bz = jnp.zeros((8, DH), _F32)
    b_out8 = _bias8(b_out)

    Wu0a, Wu0b = Wu_0[:DH], Wu_0[DH:]
    Wr0a, Wr0b = Wr_0[:DH], Wr_0[DH:]
    Wc0a, Wc0b = Wc_0[:DH], Wc_0[DH:]
    Wu1a, Wu1b = Wu_1[:DH], Wu_1[DH:]
    Wr1a, Wr1b = Wr_1[:DH], Wr_1[DH:]
    Wc1a, Wc1b = Wc_1[:DH], Wc_1[DH:]

    xpad = jnp.concatenate(
        [x_sequence, jnp.zeros((T, NPAD - N, DIN), _F32)], axis=1)
    hid0 = jnp.zeros((NPAD, DH), _F32)
    hid1 = jnp.zeros((NPAD, DH), _F32)
    outs = []
    for t in range(T):
        d0t = degp[t, 0]
        d1t = degp[t, 1]
        st = srcw[t]
        dt = dstw[t]

        hs1 = _s1_call(xpad[t], W_in_0, d0t, d1t)
        p = _prop_kernel()(hs1, st, dt)
        hs2 = _s2_call(p[0], p[1], hs1, b_in0, W_hid_0, d0t, d1t)
        p = _prop_kernel()(hs2, st, dt)
        hid0, hs3 = _gru_mid(p[0], p[1], hs2, b_hid0, hid0,
                             Wu0a, Wu0b, Wr0a, Wr0b, Wc0a, Wc0b,
                             bu0, br0, bc0, W_in_1, bz, d0t, d1t)
        p = _prop_kernel()(hs3, st, dt)
        hs4 = _s2_call(p[0], p[1], hs3, b_in1, W_hid_1, d0t, d1t)
        p = _prop_kernel()(hs4, st, dt)
        hid1, out_t = _gru_last(p[0], p[1], hs4, b_hid1, hid1,
                                Wu1a, Wu1b, Wr1a, Wr1b, Wc1a, Wc1b,
                                bu1, br1, bc1, W_out, b_out8, d0t, d1t)
        outs.append(out_t[:N])
    return jnp.stack(outs)


# speedup vs baseline: 1.2420x; 1.0663x over previous
import functools

import jax
import jax.numpy as jnp
from jax import lax
from jax.experimental import pallas as pl
from jax.experimental.pallas import tpu as pltpu
from jax.experimental.pallas import tpu_sc as plsc

N = 10000
E = 320000
T = 4
DIN = 128
DH = 64
NPAD = 10240
NC = 2
NS = 16
NW = NC * NS
EB = 128
B_CNT = 80
EPAD = NW * EB * B_CNT
ZRPS = NPAD // NS

_F32 = jnp.float32


def _deg_body(dstw, out, dst_v, oneh_v, zero_v, a0, a1, dsem):
    c = lax.axis_index("c")
    s = lax.axis_index("s")
    wid = s * NC + c
    accs = [a0, a1]

    lanes = lax.iota(jnp.int32, 16)
    e0 = jnp.where(lanes == 0, 1.0, 0.0).astype(_F32)
    z16 = jnp.zeros((16,), _F32)

    def _fill_oneh(i, carry):
        oneh_v[i, :] = e0
        return carry

    lax.fori_loop(0, EB, _fill_oneh, 0)

    def _fill_zero(i, carry):
        zero_v[i, :] = z16
        return carry

    lax.fori_loop(0, ZRPS, _fill_zero, 0)

    for phase in range(2):
        for k in range(2):
            pltpu.sync_copy(zero_v, accs[k].at[pl.ds(s * ZRPS, ZRPS)])
        plsc.subcore_barrier()

        for k in range(2):
            t = phase * 2 + k
            pltpu.sync_copy(dstw.at[t, wid], dst_v)
            acc = accs[k]

            def _b(g, carry):
                for j in range(8):
                    pltpu.async_copy(oneh_v, acc.at[dst_v.at[g * 8 + j]],
                                     dsem, add=True)
                for j in range(8):
                    pltpu.make_async_copy(oneh_v, acc.at[dst_v.at[g * 8 + j]],
                                          dsem).wait()
                return carry

            lax.fori_loop(0, B_CNT // 8, _b, 0)
        plsc.subcore_barrier()

        for k in range(2):
            t = phase * 2 + k
            pltpu.sync_copy(accs[k].at[pl.ds(s * ZRPS, ZRPS)],
                            out.at[t, c, pl.ds(s * ZRPS, ZRPS)])
        plsc.subcore_barrier()


@functools.cache
def _deg_kernel():
    return functools.partial(
        pl.kernel,
        mesh=plsc.VectorSubcoreMesh(core_axis_name="c", subcore_axis_name="s"),
        compiler_params=pltpu.CompilerParams(use_tc_tiling_on_sc=False),
        out_type=jax.ShapeDtypeStruct((T, NC, NPAD, 16), _F32),
        scratch_types=[
            pltpu.VMEM((B_CNT, EB), jnp.int32),
            pltpu.VMEM((EB, 16), _F32),
            pltpu.VMEM((ZRPS, 16), _F32),
            pltpu.VMEM_SHARED((NPAD, 16), _F32),
            pltpu.VMEM_SHARED((NPAD, 16), _F32),
            pltpu.SemaphoreType.DMA,
        ],
    )(_deg_body)


def _prop_body(hs, srcw, dstw, out, src_v, dst_v, r0, r1, r2, r3, acc,
               g0, g1, g2, g3, s0, s1, s2, s3):
    c = lax.axis_index("c")
    s = lax.axis_index("s")
    wid = s * NC + c
    rows = [r0, r1, r2, r3]
    gsems = [g0, g1, g2, g3]
    ssems = [s0, s1, s2, s3]

    z16 = jnp.zeros((16,), _F32)

    def _fill_zero(i, carry):
        for j in range(DH // 16):
            r0[i, pl.ds(j * 16, 16)] = z16
        return carry

    lax.fori_loop(0, EB, _fill_zero, 0)

    pltpu.sync_copy(srcw.at[wid], src_v)
    pltpu.sync_copy(dstw.at[wid], dst_v)

    for k in range(ZRPS // EB):
        pltpu.sync_copy(r0, acc.at[pl.ds(s * ZRPS + k * EB, EB)])

    def _fire_g(k, slot):
        pltpu.async_copy(hs.at[src_v.at[k]], rows[slot], gsems[slot])

    def _wait_g(k, slot):
        pltpu.make_async_copy(hs.at[src_v.at[k]], rows[slot],
                              gsems[slot]).wait()

    def _fire_s(k, slot):
        pltpu.async_copy(rows[slot], acc.at[dst_v.at[k]], ssems[slot],
                         add=True)

    def _wait_s(k, slot):
        pltpu.make_async_copy(rows[slot], acc.at[dst_v.at[k]],
                              ssems[slot]).wait()

    _fire_g(0, 0)
    _fire_g(1, 1)
    plsc.subcore_barrier()

    _fire_g(2, 2)
    _wait_g(0, 0)
    _fire_s(0, 0)
    _fire_g(3, 3)
    _wait_g(1, 1)
    _fire_s(1, 1)
    for k in (2, 3):
        slot = k % 4
        _wait_s(k - 2, (k + 2) % 4)
        _fire_g(k + 2, (k + 2) % 4)
        _wait_g(k, slot)
        _fire_s(k, slot)

    def _grp(g, carry):
        k0 = g * 4
        for j in range(4):
            k = k0 + j
            _wait_s(k - 2, (j + 2) % 4)
            _fire_g(k + 2, (j + 2) % 4)
            _wait_g(k, j)
            _fire_s(k, j)
        return carry

    lax.fori_loop(1, (B_CNT - 4) // 4, _grp, 0)

    for k in (B_CNT - 4, B_CNT - 3):
        slot = k % 4
        _wait_s(k - 2, (k + 2) % 4)
        _fire_g(k + 2, (k + 2) % 4)
        _wait_g(k, slot)
        _fire_s(k, slot)
    for k in (B_CNT - 2, B_CNT - 1):
        slot = k % 4
        _wait_s(k - 2, (k + 2) % 4)
        _wait_g(k, slot)
        _fire_s(k, slot)
    _wait_s(B_CNT - 2, (B_CNT - 2) % 4)
    _wait_s(B_CNT - 1, (B_CNT - 1) % 4)
    plsc.subcore_barrier()

    pltpu.sync_copy(acc.at[pl.ds(s * ZRPS, ZRPS)],
                    out.at[c, pl.ds(s * ZRPS, ZRPS)])


@functools.cache
def _prop_kernel():
    return functools.partial(
        pl.kernel,
        mesh=plsc.VectorSubcoreMesh(core_axis_name="c", subcore_axis_name="s"),
        compiler_params=pltpu.CompilerParams(use_tc_tiling_on_sc=False),
        out_type=jax.ShapeDtypeStruct((NC, NPAD, DH), _F32),
        scratch_types=[
            pltpu.VMEM((B_CNT, EB), jnp.int32),
            pltpu.VMEM((B_CNT, EB), jnp.int32),
            pltpu.VMEM((EB, DH), _F32),
            pltpu.VMEM((EB, DH), _F32),
            pltpu.VMEM((EB, DH), _F32),
            pltpu.VMEM((EB, DH), _F32),
            pltpu.VMEM_SHARED((NPAD, DH), _F32),
            pltpu.SemaphoreType.DMA,
            pltpu.SemaphoreType.DMA,
            pltpu.SemaphoreType.DMA,
            pltpu.SemaphoreType.DMA,
            pltpu.SemaphoreType.DMA,
            pltpu.SemaphoreType.DMA,
            pltpu.SemaphoreType.DMA,
            pltpu.SemaphoreType.DMA,
        ],
    )(_prop_body)


NPP = NPAD // 2
BLKP = 1024
GRID = NPP // BLKP
DD = 2 * DH


def _dinv2(d0_ref, d1_ref):
    d = d0_ref[...] + d1_ref[...]
    de = lax.rsqrt(d[:, 0:1] + 1.0)
    do = lax.rsqrt(d[:, 16:17] + 1.0)
    return jnp.concatenate([jnp.broadcast_to(de, (BLKP, DH)),
                            jnp.broadcast_to(do, (BLKP, DH))], axis=1)


def _s1_body(x_ref, w_ref, d0_ref, d1_ref, o_ref):
    dinv = _dinv2(d0_ref, d1_ref)
    h = jnp.dot(x_ref[...], w_ref[...], preferred_element_type=_F32)
    o_ref[...] = h * dinv


def _s2_body(p0_ref, p1_ref, hs_ref, b_ref, w_ref, d0_ref, d1_ref, o_ref):
    dinv = _dinv2(d0_ref, d1_ref)
    agg = p0_ref[...] + p1_ref[...] + hs_ref[...]
    cur = jnp.maximum(dinv * agg + b_ref[0:1, :], 0.0)
    o_ref[...] = jnp.dot(cur, w_ref[...], preferred_element_type=_F32) * dinv


def _gru_body(p0_ref, p1_ref, hs_ref, bh_ref, hid_ref,
              wua_ref, wub_ref, wra_ref, wrb_ref, wca_ref, wcb_ref,
              bu_ref, br_ref, bc_ref, wn_ref, bn_ref, d0_ref, d1_ref,
              hidn_ref, o_ref, *, scale_out):
    dinv = _dinv2(d0_ref, d1_ref)
    cur = dinv * (p0_ref[...] + p1_ref[...] + hs_ref[...]) + bh_ref[0:1, :]
    hid = hid_ref[...]
    dot = functools.partial(jnp.dot, preferred_element_type=_F32)
    u = jax.nn.sigmoid(dot(cur, wua_ref[...]) + dot(hid, wub_ref[...])
                       + bu_ref[0:1, :])
    r = jax.nn.sigmoid(dot(cur, wra_ref[...]) + dot(hid, wrb_ref[...])
                       + br_ref[0:1, :])
    cc = jnp.tanh(dot(cur, wca_ref[...]) + dot(r * hid, wcb_ref[...])
                  + bc_ref[0:1, :])
    hidn = (1.0 - u) * hid + u * cc
    hidn_ref[...] = hidn
    nxt = dot(hidn, wn_ref[...])
    if scale_out:
        o_ref[...] = nxt * dinv + bn_ref[0:1, :]
    else:
        o_ref[...] = nxt + bn_ref[0:1, :]


def _row_spec(cols):
    return pl.BlockSpec((BLKP, cols), lambda i: (i, 0))


def _full_spec(rows, cols):
    return pl.BlockSpec((rows, cols), lambda i: (0, 0))


_s1_call = pl.pallas_call(
    _s1_body,
    grid=(GRID,),
    in_specs=[_row_spec(2 * DIN), _full_spec(2 * DIN, DD), _row_spec(32),
              _row_spec(32)],
    out_specs=_row_spec(DD),
    out_shape=jax.ShapeDtypeStruct((NPP, DD), _F32),
)

_s2_call = pl.pallas_call(
    _s2_body,
    grid=(GRID,),
    in_specs=[_row_spec(DD), _row_spec(DD), _row_spec(DD),
              _full_spec(8, DD), _full_spec(DD, DD), _row_spec(32),
              _row_spec(32)],
    out_specs=_row_spec(DD),
    out_shape=jax.ShapeDtypeStruct((NPP, DD), _F32),
)


def _gru_call(scale_out):
    return pl.pallas_call(
        functools.partial(_gru_body, scale_out=scale_out),
        grid=(GRID,),
        in_specs=[_row_spec(DD), _row_spec(DD), _row_spec(DD),
                  _full_spec(8, DD), _row_spec(DD)]
                 + [_full_spec(DD, DD)] * 6
                 + [_full_spec(8, DD)] * 3
                 + [_full_spec(DD, DD), _full_spec(8, DD),
                    _row_spec(32), _row_spec(32)],
        out_specs=[_row_spec(DD), _row_spec(DD)],
        out_shape=[jax.ShapeDtypeStruct((NPP, DD), _F32),
                   jax.ShapeDtypeStruct((NPP, DD), _F32)],
    )


_gru_mid = _gru_call(True)
_gru_last = _gru_call(False)


def _bias8(b):
    bb = jnp.concatenate([b, b]).reshape(1, DD)
    return jnp.broadcast_to(bb, (8, DD))


def _bdiag(w):
    z = jnp.zeros(w.shape, _F32)
    return jnp.concatenate(
        [jnp.concatenate([w, z], axis=1), jnp.concatenate([z, w], axis=1)],
        axis=0)


def kernel(x_sequence, edge_index_sequence,
           W_in_0, b_in_0, W_hid_0, b_hid_0, Wu_0, bu_0, Wr_0, br_0,
           Wc_0, bc_0,
           W_in_1, b_in_1, W_hid_1, b_hid_1, Wu_1, bu_1, Wr_1, br_1,
           Wc_1, bc_1, W_out, b_out):
    src = edge_index_sequence[:, 0, :]
    dst = edge_index_sequence[:, 1, :]
    npd = EPAD - E
    pad_src = jnp.broadcast_to(
        (jnp.arange(npd, dtype=jnp.int32) % 240)[None, :], (T, npd))
    pad_dst = jnp.broadcast_to(
        (N + jnp.arange(npd, dtype=jnp.int32) % (NPAD - N))[None, :], (T, npd))
    srcw = jnp.concatenate([src, pad_src], axis=1).reshape(T, NW, B_CNT, EB)
    dstw = jnp.concatenate([dst, pad_dst], axis=1).reshape(T, NW, B_CNT, EB)

    degp = _deg_kernel()(dstw)
    degpp = degp.reshape(T, NC, NPP, 32)

    b_in0 = _bias8(b_in_0)
    b_hid0 = _bias8(b_hid_0)
    b_in1 = _bias8(b_in_1)
    b_hid1 = _bias8(b_hid_1)
    bu0, br0, bc0 = _bias8(bu_0), _bias8(br_0), _bias8(bc_0)
    bu1, br1, bc1 = _bias8(bu_1), _bias8(br_1), _bias8(bc_1)
    bz = jnp.zeros((8, DD), _F32)
    b_out8 = _bias8(b_out)

    Wi0 = _bdiag(W_in_0)
    Wh0 = _bdiag(W_hid_0)
    Wi1 = _bdiag(W_in_1)
    Wh1 = _bdiag(W_hid_1)
    Wo = _bdiag(W_out)
    Wu0a, Wu0b = _bdiag(Wu_0[:DH]), _bdiag(Wu_0[DH:])
    Wr0a, Wr0b = _bdiag(Wr_0[:DH]), _bdiag(Wr_0[DH:])
    Wc0a, Wc0b = _bdiag(Wc_0[:DH]), _bdiag(Wc_0[DH:])
    Wu1a, Wu1b = _bdiag(Wu_1[:DH]), _bdiag(Wu_1[DH:])
    Wr1a, Wr1b = _bdiag(Wr_1[:DH]), _bdiag(Wr_1[DH:])
    Wc1a, Wc1b = _bdiag(Wc_1[:DH]), _bdiag(Wc_1[DH:])

    xpad = jnp.concatenate(
        [x_sequence, jnp.zeros((T, NPAD - N, DIN), _F32)],
        axis=1).reshape(T, NPP, 2 * DIN)
    hid0 = jnp.zeros((NPP, DD), _F32)
    hid1 = jnp.zeros((NPP, DD), _F32)
    outs = []
    for t in range(T):
        d0t = degpp[t, 0]
        d1t = degpp[t, 1]
        st = srcw[t]
        dt = dstw[t]

        def _prop(a):
            pp = _prop_kernel()(a.reshape(NPAD, DH), st, dt)
            pp = pp.reshape(NC, NPP, DD)
            return pp[0], pp[1]

        hs1 = _s1_call(xpad[t], Wi0, d0t, d1t)
        p0, p1 = _prop(hs1)
        hs2 = _s2_call(p0, p1, hs1, b_in0, Wh0, d0t, d1t)
        p0, p1 = _prop(hs2)
        hid0, hs3 = _gru_mid(p0, p1, hs2, b_hid0, hid0,
                             Wu0a, Wu0b, Wr0a, Wr0b, Wc0a, Wc0b,
                             bu0, br0, bc0, Wi1, bz, d0t, d1t)
        p0, p1 = _prop(hs3)
        hs4 = _s2_call(p0, p1, hs3, b_in1, Wh1, d0t, d1t)
        p0, p1 = _prop(hs4)
        hid1, out_t = _gru_last(p0, p1, hs4, b_hid1, hid1,
                                Wu1a, Wu1b, Wr1a, Wr1b, Wc1a, Wc1b,
                                bu1, br1, bc1, Wo, b_out8, d0t, d1t)
        outs.append(out_t.reshape(NPAD, DH)[:N])
    return jnp.stack(outs)
